# baseline (speedup 1.0000x reference)
import functools

import jax
import jax.numpy as jnp
from jax import lax
from jax.experimental import pallas as pl
from jax.experimental.pallas import tpu as pltpu
from jax.experimental.pallas import tpu_sc as plsc

B, S, D = 1024, 200, 64
N = B * S
NUM_TYPES = 4
POS_ROWS = S
EPS = 1e-12
NC, NS = 2, 16
NW = NC * NS
TPW = N // NW
C = 128
NCHUNK = TPW // C


def _sc_body(w1, w2, w3, wpos, wtype, gb,
             i1, i2, i3, ip, it, out,
             posbuf, typebuf, gbbuf,
             idx1, idx2, idx3, idxp, idxt,
             r1, r2, r3, outv,
             semg0, semg1, semo0, semo1):
  wid = lax.axis_index("s") * NC + lax.axis_index("c")
  base = wid * TPW
  semg = [semg0, semg1]
  semo = [semo0, semo1]

  pltpu.sync_copy(wpos, posbuf)
  pltpu.sync_copy(wtype, typebuf)
  pltpu.sync_copy(gb, gbbuf)

  iota = lax.iota(jnp.int32, 16)
  cols = [iota + (k * 16) for k in range(4)]
  gvecs = [gbbuf[0, k * 16:(k + 1) * 16] for k in range(4)]
  bvecs = [gbbuf[1, k * 16:(k + 1) * 16] for k in range(4)]

  def copy_ids(j, slot):
    off = base + j * C
    pltpu.sync_copy(i1.at[pl.ds(off, C)], idx1.at[slot])
    pltpu.sync_copy(i2.at[pl.ds(off, C)], idx2.at[slot])
    pltpu.sync_copy(i3.at[pl.ds(off, C)], idx3.at[slot])
    pltpu.sync_copy(ip.at[pl.ds(off, C)], idxp.at[slot])
    pltpu.sync_copy(it.at[pl.ds(off, C)], idxt.at[slot])

  def fire_gathers(slot):
    pltpu.make_async_copy(w1.at[idx1.at[slot]], r1.at[slot], semg[slot]).start()
    pltpu.make_async_copy(w2.at[idx2.at[slot]], r2.at[slot], semg[slot]).start()
    pltpu.make_async_copy(w3.at[idx3.at[slot]], r3.at[slot], semg[slot]).start()

  def wait_gathers(slot):
    pltpu.make_async_copy(w1.at[idx1.at[slot]], r1.at[slot], semg[slot]).wait()
    pltpu.make_async_copy(w2.at[idx2.at[slot]], r2.at[slot], semg[slot]).wait()
    pltpu.make_async_copy(w3.at[idx3.at[slot]], r3.at[slot], semg[slot]).wait()

  def out_copy(j, slot):
    return pltpu.make_async_copy(
        outv.at[slot], out.at[pl.ds(base + j * C, C)], semo[slot])

  copy_ids(0, 0)
  fire_gathers(0)

  @pl.loop(0, NCHUNK, step=2)
  def chunk_pair(j0):
    for slot in range(2):
      j = j0 + slot
      other = 1 - slot

      with jax.named_scope("stage_next"):
        @pl.when(j + 1 < NCHUNK)
        def _stage():
          copy_ids(j + 1, other)
          fire_gathers(other)

      with jax.named_scope("gather_wait"):
        wait_gathers(slot)
      @pl.when(j >= 2)
      def _drain():
        out_copy(j - 2, slot).wait()

      def tok(i, carry):
        row = jnp.full((16,), i, dtype=jnp.int32)
        p = plsc.load_gather(idxp.at[slot], [row])
        t = plsc.load_gather(idxt.at[slot], [row])
        acc = []
        for k in range(4):
          v1 = plsc.load_gather(r1.at[slot], [row, cols[k]])
          v2 = plsc.load_gather(r2.at[slot], [row, cols[k]])
          v3 = plsc.load_gather(r3.at[slot], [row, cols[k]])
          vp = plsc.load_gather(posbuf, [p, cols[k]])
          vt = plsc.load_gather(typebuf, [t, cols[k]])
          acc.append(((v1 + v2) + (v3 + vp)) + vt)
        s = (acc[0] + acc[1]) + (acc[2] + acc[3])
        tot = jnp.sum(s)
        ssq = jnp.sum((acc[0] * acc[0] + acc[1] * acc[1]) +
                      (acc[2] * acc[2] + acc[3] * acc[3]))
        mean = tot * (1.0 / 64.0)
        var = ssq * (1.0 / 64.0) - mean * mean
        x = jnp.maximum(var, 0.0) + EPS
        xi = lax.bitcast_convert_type(x, jnp.int32)
        yi = jnp.int32(0x5F3759DF) - (xi >> 1)
        y = lax.bitcast_convert_type(yi, jnp.float32)
        hx = 0.5 * x
        for _ in range(3):
          y = y * (1.5 - hx * (y * y))
        mean_v = jnp.full((16,), mean, dtype=jnp.float32)
        rstd_v = jnp.full((16,), y, dtype=jnp.float32)
        for k in range(4):
          o = (acc[k] - mean_v) * rstd_v * gvecs[k] + bvecs[k]
          plsc.store_scatter(outv.at[slot], [row, cols[k]], o)
        return carry

      with jax.named_scope("tok_loop"):
        lax.fori_loop(0, C, tok, 0, unroll=8)
      out_copy(j, slot).start()

  out_copy(NCHUNK - 2, 0).wait()
  out_copy(NCHUNK - 1, 1).wait()


@jax.jit
def kernel(l1_ids, l2_ids, l3_ids, positions, token_types,
           W1, W2, W3, Wpos, Wtype, gamma, beta):
  i1 = l1_ids.reshape(-1).astype(jnp.int32)
  i2 = l2_ids.reshape(-1).astype(jnp.int32)
  i3 = l3_ids.reshape(-1).astype(jnp.int32)
  ipos = positions.reshape(-1).astype(jnp.int32)
  ityp = token_types.reshape(-1).astype(jnp.int32)
  gb = jnp.stack([gamma, beta]).astype(jnp.float32)
  wpos = Wpos[:POS_ROWS]

  mesh = plsc.VectorSubcoreMesh(core_axis_name="c", subcore_axis_name="s",
                                num_cores=NC, num_subcores=NS)
  run = pl.kernel(
      _sc_body,
      out_type=jax.ShapeDtypeStruct((N, D), jnp.float32),
      mesh=mesh,
      compiler_params=pltpu.CompilerParams(needs_layout_passes=False,
                                           use_tc_tiling_on_sc=False),
      scratch_types=[
          pltpu.VMEM((POS_ROWS, D), jnp.float32),
          pltpu.VMEM((NUM_TYPES, D), jnp.float32),
          pltpu.VMEM((2, D), jnp.float32),
          pltpu.VMEM((2, C), jnp.int32),
          pltpu.VMEM((2, C), jnp.int32),
          pltpu.VMEM((2, C), jnp.int32),
          pltpu.VMEM((2, C), jnp.int32),
          pltpu.VMEM((2, C), jnp.int32),
          pltpu.VMEM((2, C, D), jnp.float32),
          pltpu.VMEM((2, C, D), jnp.float32),
          pltpu.VMEM((2, C, D), jnp.float32),
          pltpu.VMEM((2, C, D), jnp.float32),
          pltpu.SemaphoreType.DMA,
          pltpu.SemaphoreType.DMA,
          pltpu.SemaphoreType.DMA,
          pltpu.SemaphoreType.DMA,
      ],
  )
  out = run(W1, W2, W3, wpos, Wtype, gb, i1, i2, i3, ipos, ityp)
  return out.reshape(B, S, D)


# speedup vs baseline: 1.2247x; 1.2247x over previous
import functools

import jax
import jax.numpy as jnp
from jax import lax
from jax.experimental import pallas as pl
from jax.experimental.pallas import tpu as pltpu
from jax.experimental.pallas import tpu_sc as plsc

B, S, D = 1024, 200, 64
N = B * S
NUM_TYPES = 4
POS_ROWS = S
EPS = 1e-12
NC, NS = 2, 16
NW = NC * NS
TPW = N // NW
C = 128
NCHUNK = TPW // C


def _sc_body(w1, w2, w3, wpos, wtype, gb,
             i1, i2, i3, ip, it, out,
             posbuf, typebuf, gbbuf,
             idx1, idx2, idx3, idxp, idxt,
             r1, r2, r3, outv,
             semg0, semg1, semo0, semo1, semi0, semi1):
  wid = lax.axis_index("s") * NC + lax.axis_index("c")
  base = wid * TPW
  semg = [semg0, semg1]
  semo = [semo0, semo1]
  semi = [semi0, semi1]

  pltpu.sync_copy(wpos, posbuf)
  pltpu.sync_copy(wtype, typebuf)
  pltpu.sync_copy(gb, gbbuf)

  gvecs = [gbbuf[0, k * 16:(k + 1) * 16] for k in range(4)]
  bvecs = [gbbuf[1, k * 16:(k + 1) * 16] for k in range(4)]

  def id_copies(j, slot):
    off = base + j * C
    return [
        pltpu.make_async_copy(i1.at[pl.ds(off, C)], idx1.at[slot], semi[slot]),
        pltpu.make_async_copy(i2.at[pl.ds(off, C)], idx2.at[slot], semi[slot]),
        pltpu.make_async_copy(i3.at[pl.ds(off, C)], idx3.at[slot], semi[slot]),
        pltpu.make_async_copy(ip.at[pl.ds(off, C)], idxp.at[slot], semi[slot]),
        pltpu.make_async_copy(it.at[pl.ds(off, C)], idxt.at[slot], semi[slot]),
    ]

  def fire_ids(j, slot):
    for c in id_copies(j, slot):
      c.start()

  def wait_ids(j, slot):
    for c in id_copies(j, slot):
      c.wait()

  def gather_copies(slot):
    return [
        pltpu.make_async_copy(w1.at[idx1.at[slot]], r1.at[slot], semg[slot]),
        pltpu.make_async_copy(w2.at[idx2.at[slot]], r2.at[slot], semg[slot]),
        pltpu.make_async_copy(w3.at[idx3.at[slot]], r3.at[slot], semg[slot]),
    ]

  def fire_gathers(slot):
    for c in gather_copies(slot):
      c.start()

  def wait_gathers(slot):
    for c in gather_copies(slot):
      c.wait()

  def out_copy(j, slot):
    return pltpu.make_async_copy(
        outv.at[slot], out.at[pl.ds(base + j * C, C)], semo[slot])

  fire_ids(0, 0)
  wait_ids(0, 0)
  fire_gathers(0)
  fire_ids(1, 1)

  @pl.loop(0, NCHUNK, step=2)
  def chunk_pair(j0):
    for slot in range(2):
      j = j0 + slot
      other = 1 - slot

      @pl.when(j + 1 < NCHUNK)
      def _stage():
        wait_ids(j + 1, other)
        fire_gathers(other)

      wait_gathers(slot)
      @pl.when(j + 2 < NCHUNK)
      def _ids():
        fire_ids(j + 2, slot)

      @pl.when(j >= 2)
      def _drain():
        out_copy(j - 2, slot).wait()

      def grp(g, carry):
        g16 = g * 16
        pg = idxp[slot, pl.ds(g16, 16)]
        tg = idxt[slot, pl.ds(g16, 16)]
        for ii in range(16):
          i = g16 + ii
          p = pg[ii]
          t = tg[ii]
          acc = []
          for k in range(4):
            sl = pl.ds(k * 16, 16)
            v1 = r1[slot, i, sl]
            v2 = r2[slot, i, sl]
            v3 = r3[slot, i, sl]
            vp = posbuf[p, sl]
            vt = typebuf[t, sl]
            acc.append(((v1 + v2) + (v3 + vp)) + vt)
          s = (acc[0] + acc[1]) + (acc[2] + acc[3])
          tot = jnp.sum(s)
          ssq = jnp.sum((acc[0] * acc[0] + acc[1] * acc[1]) +
                        (acc[2] * acc[2] + acc[3] * acc[3]))
          mean = tot * (1.0 / 64.0)
          var = ssq * (1.0 / 64.0) - mean * mean
          x = jnp.maximum(var, 0.0) + EPS
          xi = lax.bitcast_convert_type(x, jnp.int32)
          yi = jnp.int32(0x5F3759DF) - (xi >> 1)
          y = lax.bitcast_convert_type(yi, jnp.float32)
          hx = 0.5 * x
          for _ in range(3):
            y = y * (1.5 - hx * (y * y))
          mean_v = jnp.full((16,), mean, dtype=jnp.float32)
          rstd_v = jnp.full((16,), y, dtype=jnp.float32)
          for k in range(4):
            o = (acc[k] - mean_v) * rstd_v * gvecs[k] + bvecs[k]
            outv[slot, i, pl.ds(k * 16, 16)] = o
        return carry

      with jax.named_scope("tok_loop"):
        lax.fori_loop(0, C // 16, grp, 0, unroll=1)
      out_copy(j, slot).start()

  out_copy(NCHUNK - 2, 0).wait()
  out_copy(NCHUNK - 1, 1).wait()


@jax.jit
def kernel(l1_ids, l2_ids, l3_ids, positions, token_types,
           W1, W2, W3, Wpos, Wtype, gamma, beta):
  i1 = l1_ids.reshape(-1).astype(jnp.int32)
  i2 = l2_ids.reshape(-1).astype(jnp.int32)
  i3 = l3_ids.reshape(-1).astype(jnp.int32)
  ipos = positions.reshape(-1).astype(jnp.int32)
  ityp = token_types.reshape(-1).astype(jnp.int32)
  gb = jnp.stack([gamma, beta]).astype(jnp.float32)
  wpos = Wpos[:POS_ROWS]

  mesh = plsc.VectorSubcoreMesh(core_axis_name="c", subcore_axis_name="s",
                                num_cores=NC, num_subcores=NS)
  run = pl.kernel(
      _sc_body,
      out_type=jax.ShapeDtypeStruct((N, D), jnp.float32),
      mesh=mesh,
      compiler_params=pltpu.CompilerParams(needs_layout_passes=False,
                                           use_tc_tiling_on_sc=False),
      scratch_types=[
          pltpu.VMEM((POS_ROWS, D), jnp.float32),
          pltpu.VMEM((NUM_TYPES, D), jnp.float32),
          pltpu.VMEM((2, D), jnp.float32),
          pltpu.VMEM((2, C), jnp.int32),
          pltpu.VMEM((2, C), jnp.int32),
          pltpu.VMEM((2, C), jnp.int32),
          pltpu.VMEM((2, C), jnp.int32),
          pltpu.VMEM((2, C), jnp.int32),
          pltpu.VMEM((2, C, D), jnp.float32),
          pltpu.VMEM((2, C, D), jnp.float32),
          pltpu.VMEM((2, C, D), jnp.float32),
          pltpu.VMEM((2, C, D), jnp.float32),
          pltpu.SemaphoreType.DMA,
          pltpu.SemaphoreType.DMA,
          pltpu.SemaphoreType.DMA,
          pltpu.SemaphoreType.DMA,
          pltpu.SemaphoreType.DMA,
          pltpu.SemaphoreType.DMA,
      ],
  )
  out = run(W1, W2, W3, wpos, Wtype, gb, i1, i2, i3, ipos, ityp)
  return out.reshape(B, S, D)
